# Initial kernel scaffold; baseline (speedup 1.0000x reference)
import jax
import jax.numpy as jnp
from jax.experimental import pallas as pl


def kernel(pred, target):
    raise NotImplementedError("write your pallas kernel here")


# speedup vs baseline: 1.9939x; 1.9939x over previous
import functools

import jax
import jax.numpy as jnp
from jax import lax
from jax.experimental import pallas as pl
from jax.experimental.pallas import tpu as pltpu

_SMOOTHING = 0.1
_CONFIDENCE = 1.0 - _SMOOTHING

_R = 1024
_C = 100000
_CB = 2048
_NB = (_C + _CB - 1) // _CB


def _tc_body(pred_ref, tgt_ref, out_ref, acc_s, acc_m):
    j = pl.program_id(0)
    p = pred_ref[...]
    cols = lax.broadcasted_iota(jnp.int32, (_R, _CB), 1) + j * _CB
    m = jnp.sum(jnp.where(cols == tgt_ref[...], p, 0.0))

    @pl.when(j == 0)
    def _init():
        acc_s[0] = 0.0
        acc_m[0] = 0.0

    @pl.when(j < _NB - 1)
    def _mid():
        acc_s[0] += jnp.sum(p)

    @pl.when(j == _NB - 1)
    def _last():
        s = jnp.sum(jnp.where(cols < _C, p, 0.0))
        eps = _SMOOTHING / (_C - 1)
        s_all = acc_s[0] + s
        s_tgt = acc_m[0] + m
        out_ref[0] = (-eps * s_all - (_CONFIDENCE - eps) * s_tgt) / _R

    @pl.when(j < _NB - 1)
    def _mid_m():
        acc_m[0] += m


@functools.partial(jax.jit, static_argnames=("interpret",))
def _loss(pred, target, interpret=False):
    tgt2d = target.astype(jnp.int32).reshape(_R, 1)
    out = pl.pallas_call(
        _tc_body,
        grid=(_NB,),
        in_specs=[
            pl.BlockSpec((_R, _CB), lambda j: (0, j)),
            pl.BlockSpec((_R, 1), lambda j: (0, 0)),
        ],
        out_specs=pl.BlockSpec(memory_space=pltpu.SMEM),
        out_shape=jax.ShapeDtypeStruct((1,), jnp.float32),
        scratch_shapes=[
            pltpu.SMEM((1,), jnp.float32),
            pltpu.SMEM((1,), jnp.float32),
        ],
        interpret=interpret,
    )(pred, tgt2d)
    return out[0]


def kernel(pred, target):
    return _loss(pred, target)
